# Initial kernel scaffold; baseline (speedup 1.0000x reference)
import jax
import jax.numpy as jnp
from jax.experimental import pallas as pl


def kernel(x, edge_index, edge_attr, params):
    raise NotImplementedError("write your pallas kernel here")


# speedup vs baseline: 2.5698x; 2.5698x over previous
import functools

import jax
import jax.numpy as jnp
from jax.experimental import pallas as pl

_N = 10000
_E = 320000
_K = 5
_IN = 128
_HID = 128
_T = 2

_BLK = 2000


def _elu(v):
    return jnp.where(v > 0, v, jnp.exp(v) - 1.0)


def _init_body(x_ref, w1_ref, b1_ref, w2_ref, b2_ref, out_ref):
    y = _elu(jnp.dot(x_ref[...], w1_ref[...], preferred_element_type=jnp.float32)
             + b1_ref[...])
    out_ref[...] = jnp.tanh(
        jnp.dot(y, w2_ref[...], preferred_element_type=jnp.float32) + b2_ref[...])


def _rk4_body(h_ref, wi_ref, bi_ref, wh_ref, bh_ref, wo_ref, bo_ref, out_ref):
    wi = wi_ref[...]
    bi = bi_ref[...]
    wh = wh_ref[...]
    bh = bh_ref[...]
    wo = wo_ref[...]
    bo = bo_ref[...]

    def f(v):
        y = _elu(jnp.dot(v, wi, preferred_element_type=jnp.float32) + bi)
        y = _elu(jnp.dot(y, wh, preferred_element_type=jnp.float32) + bh)
        return jnp.tanh(jnp.dot(y, wo, preferred_element_type=jnp.float32) + bo)

    h = h_ref[...]
    k1 = f(h)
    k2 = f(h + 0.5 * k1)
    k3 = f(h + 0.5 * k2)
    k4 = f(h + k3)
    out_ref[...] = h + (k1 + 2.0 * k2 + 2.0 * k3 + k4) * (1.0 / 6.0)


def _gates_body(acc_ref, inv_ref, xt_ref, hid_ref, wbx_ref, wbh_ref,
                rootx_ref, rooth_ref, bx_ref, bh_ref, out_ref):
    acc = acc_ref[...]
    b = acc.shape[0]
    accx = acc[:, :, :_IN].reshape(b, _K * _IN)
    acch = acc[:, :, _IN:].reshape(b, _K * _HID)
    inv = inv_ref[...]
    ax = (jnp.dot(accx, wbx_ref[...], preferred_element_type=jnp.float32) * inv
          + jnp.dot(xt_ref[...], rootx_ref[...], preferred_element_type=jnp.float32)
          + bx_ref[...])
    ah = (jnp.dot(acch, wbh_ref[...], preferred_element_type=jnp.float32) * inv
          + jnp.dot(hid_ref[...], rooth_ref[...], preferred_element_type=jnp.float32)
          + bh_ref[...])
    xr = ax[:, :_HID]
    xz = ax[:, _HID:2 * _HID]
    xn = ax[:, 2 * _HID:]
    hr = ah[:, :_HID]
    hz = ah[:, _HID:]
    r = jax.nn.sigmoid(xr + hr)
    z = jax.nn.sigmoid(xz + hz)
    n = jnp.tanh(xn + r * hr)
    hid = hid_ref[...]
    out_ref[...] = (1.0 - z) * n + z * hid


def _row_spec(width):
    return pl.BlockSpec((_BLK, width), lambda i: (i, 0))


def _full_spec(shape):
    nd = len(shape)
    return pl.BlockSpec(shape, lambda i: (0,) * nd)


@jax.jit
def _run(x, edge_index, edge_attr, params):
    p = params
    src = edge_index[0]
    dst = edge_index[1]

    u = edge_attr[:, 0] * (_K - 1)
    lo = jnp.clip(jnp.floor(u), 0.0, _K - 2)
    frac = u - lo
    loi = lo.astype(jnp.int32)
    idx0 = dst * _K + loi
    idx1 = idx0 + 1

    deg = jax.ops.segment_sum(jnp.ones((_E,), jnp.float32), dst, num_segments=_N)
    invdeg = (1.0 / jnp.maximum(deg, 1.0))[:, None]

    wbx = jnp.concatenate([p['xr_W'], p['xz_W'], p['xn_W']], axis=2)
    wbx = wbx.reshape(_K * _IN, 3 * _HID)
    wbh = jnp.concatenate([p['hr_W'], p['hz_W']], axis=2)
    wbh = wbh.reshape(_K * _HID, 2 * _HID)
    rootx = jnp.concatenate([p['xr_root'], p['xz_root'], p['xn_root']], axis=1)
    rooth = jnp.concatenate([p['hr_root'], p['hz_root']], axis=1)
    bx = jnp.concatenate([p['xr_b'], p['xz_b'], p['xn_b']])[None, :]
    bh = jnp.concatenate([p['hr_b'], p['hz_b']])[None, :]

    grid = (_N // _BLK,)
    f32 = jnp.float32

    init_call = pl.pallas_call(
        _init_body,
        grid=grid,
        in_specs=[_row_spec(_IN), _full_spec((_IN, 2 * _HID)),
                  _full_spec((1, 2 * _HID)), _full_spec((2 * _HID, _HID)),
                  _full_spec((1, _HID))],
        out_specs=_row_spec(_HID),
        out_shape=jax.ShapeDtypeStruct((_N, _HID), f32),
    )
    rk4_call = pl.pallas_call(
        _rk4_body,
        grid=grid,
        in_specs=[_row_spec(_HID),
                  _full_spec((_HID, 2 * _HID)), _full_spec((1, 2 * _HID)),
                  _full_spec((2 * _HID, 2 * _HID)), _full_spec((1, 2 * _HID)),
                  _full_spec((2 * _HID, _HID)), _full_spec((1, _HID))],
        out_specs=_row_spec(_HID),
        out_shape=jax.ShapeDtypeStruct((_N, _HID), f32),
    )
    gates_call = pl.pallas_call(
        _gates_body,
        grid=grid,
        in_specs=[pl.BlockSpec((_BLK, _K, 2 * _HID), lambda i: (i, 0, 0)),
                  _row_spec(1), _row_spec(_IN), _row_spec(_HID),
                  _full_spec((_K * _IN, 3 * _HID)),
                  _full_spec((_K * _HID, 2 * _HID)),
                  _full_spec((_IN, 3 * _HID)), _full_spec((_HID, 2 * _HID)),
                  _full_spec((1, 3 * _HID)), _full_spec((1, 2 * _HID))],
        out_specs=_row_spec(_HID),
        out_shape=jax.ShapeDtypeStruct((_N, _HID), f32),
    )

    xt = jnp.transpose(x[0], (2, 0, 1))

    last_h = init_call(xt[_T - 1],
                       p['init_g1_W'].T, p['init_g1_b'][None, :],
                       p['init_g2_W'].T, p['init_g2_b'][None, :])

    wi = p['ode_in_W'].T
    bi = p['ode_in_b'][None, :]
    wh = p['ode_h0_W'].T
    bhh = p['ode_h0_b'][None, :]
    wo = p['ode_out_W'].T
    bo = p['ode_out_b'][None, :]

    for t in range(_T - 1, -1, -1):
        hidden = rk4_call(last_h, wi, bi, wh, bhh, wo, bo)
        x_t = xt[t]
        vcat = jnp.concatenate([x_t, hidden], axis=1)
        g = jnp.take(vcat, src, axis=0)
        g1 = frac[:, None] * g
        g0 = g - g1
        acc = (jax.ops.segment_sum(g0, idx0, num_segments=_N * _K)
               + jax.ops.segment_sum(g1, idx1, num_segments=_N * _K))
        acc = acc.reshape(_N, _K, 2 * _HID)
        last_h = gates_call(acc, invdeg, x_t, hidden,
                            wbx, wbh, rootx, rooth, bx, bh)
    return last_h


def kernel(x, edge_index, edge_attr, params):
    return _run(x, edge_index, edge_attr, params)
